# baseline (speedup 1.0000x reference)
import functools

import jax
import jax.numpy as jnp
from jax import lax
from jax.experimental import pallas as pl
from jax.experimental.pallas import tpu as pltpu
from jax.experimental.pallas import tpu_sc as plsc

N_NODES = 100000
N_PAD = 100352
ROWS_N = 784
E_EDGES = 6400000
E_PAD = 6553600
WIN = 8192
WING = 4096
NC, NS = 2, 16
NW = NC * NS

_mesh = plsc.VectorSubcoreMesh(
    core_axis_name="c", subcore_axis_name="s", num_cores=NC, num_subcores=NS)


def _gather_loop(tab_v, idx_v, upd_v, n):
    @pl.loop(0, n // 16, unroll=8)
    def step(j):
        idx16 = idx_v[pl.ds(j * 16, 16)]
        upd_v[pl.ds(j * 16, 16)] = plsc.load_gather(tab_v, [idx16])


@functools.partial(
    pl.kernel,
    out_type=jax.ShapeDtypeStruct((NC, N_PAD), jnp.float32),
    mesh=_mesh,
    scratch_types=[
        pltpu.VMEM((WIN,), jnp.int32),
        pltpu.VMEM((WIN,), jnp.float32),
        pltpu.VMEM_SHARED((N_PAD,), jnp.float32),
    ],
)
def _sc_count(dst_hbm, zeros_hbm, ones_hbm, out_hbm, idx_v, upd_v, acc_sh):
    cid = lax.axis_index("c")
    sid = lax.axis_index("s")
    wid = sid * NC + cid

    @pl.when(sid == 0)
    def _():
        pltpu.sync_copy(zeros_hbm, acc_sh)

    pltpu.sync_copy(ones_hbm, upd_v)
    plsc.subcore_barrier()

    def body(w, carry):
        e0 = wid * (E_PAD // NW) + w * WIN
        pltpu.sync_copy(dst_hbm.at[pl.ds(e0, WIN)], idx_v)
        pltpu.sync_copy(upd_v, acc_sh.at[idx_v], add=True)
        return carry

    lax.fori_loop(0, E_PAD // NW // WIN, body, 0)
    plsc.subcore_barrier()
    sl = N_PAD // NS
    pltpu.sync_copy(acc_sh.at[pl.ds(sid * sl, sl)],
                    out_hbm.at[cid, pl.ds(sid * sl, sl)])


@functools.partial(
    pl.kernel,
    out_type=jax.ShapeDtypeStruct((NC, N_PAD), jnp.float32),
    mesh=_mesh,
    compiler_params=pltpu.CompilerParams(needs_layout_passes=False),
    scratch_types=[
        pltpu.VMEM((N_PAD,), jnp.float32),
        pltpu.VMEM((WING,), jnp.int32),
        pltpu.VMEM((WING,), jnp.int32),
        pltpu.VMEM((WING,), jnp.float32),
        pltpu.VMEM_SHARED((N_PAD,), jnp.float32),
    ],
)
def _sc_seg_scalar(src_hbm, dst_hbm, tab_hbm, zeros_hbm, out_hbm,
                   tab_v, sidx_v, didx_v, upd_v, acc_sh):
    cid = lax.axis_index("c")
    sid = lax.axis_index("s")
    wid = sid * NC + cid

    @pl.when(sid == 0)
    def _():
        pltpu.sync_copy(zeros_hbm, acc_sh)

    pltpu.sync_copy(tab_hbm, tab_v)
    plsc.subcore_barrier()

    def body(w, carry):
        e0 = wid * (E_PAD // NW) + w * WING
        pltpu.sync_copy(src_hbm.at[pl.ds(e0, WING)], sidx_v)
        pltpu.sync_copy(dst_hbm.at[pl.ds(e0, WING)], didx_v)
        _gather_loop(tab_v, sidx_v, upd_v, WING)
        pltpu.sync_copy(upd_v, acc_sh.at[didx_v], add=True)
        return carry

    lax.fori_loop(0, E_PAD // NW // WING, body, 0)
    plsc.subcore_barrier()
    sl = N_PAD // NS
    pltpu.sync_copy(acc_sh.at[pl.ds(sid * sl, sl)],
                    out_hbm.at[cid, pl.ds(sid * sl, sl)])


@functools.partial(
    pl.kernel,
    out_type=jax.ShapeDtypeStruct((NC, N_PAD), jnp.float32),
    mesh=_mesh,
    compiler_params=pltpu.CompilerParams(needs_layout_passes=False),
    scratch_types=[
        pltpu.VMEM((N_PAD,), jnp.float32),
        pltpu.VMEM((WING,), jnp.int32),
        pltpu.VMEM((WING,), jnp.int32),
        pltpu.VMEM((WING,), jnp.float32),
        pltpu.VMEM_SHARED((N_PAD,), jnp.float32),
    ],
)
def _sc_seg_feat(src_hbm, dst_hbm, qtab_hbm, zeros_hbm, out_hbm,
                 tab_v, sidx_v, didx_v, upd_v, acc_sh):
    cid = lax.axis_index("c")
    sid = lax.axis_index("s")

    @pl.when(sid == 0)
    def _():
        pltpu.sync_copy(zeros_hbm, acc_sh)

    pltpu.sync_copy(qtab_hbm.at[cid], tab_v)
    plsc.subcore_barrier()

    def body(w, carry):
        e0 = sid * (E_PAD // NS) + w * WING
        pltpu.sync_copy(src_hbm.at[pl.ds(e0, WING)], sidx_v)
        pltpu.sync_copy(dst_hbm.at[pl.ds(e0, WING)], didx_v)
        _gather_loop(tab_v, sidx_v, upd_v, WING)
        pltpu.sync_copy(upd_v, acc_sh.at[didx_v], add=True)
        return carry

    lax.fori_loop(0, E_PAD // NS // WING, body, 0)
    plsc.subcore_barrier()
    sl = N_PAD // NS
    pltpu.sync_copy(acc_sh.at[pl.ds(sid * sl, sl)],
                    out_hbm.at[cid, pl.ds(sid * sl, sl)])


def _tc_prep_body(cnt0, cnt1, xr, dinv_o, p_o):
    deg = cnt0[...] + cnt1[...] + 1.0
    dinv = lax.rsqrt(deg)
    dinv_o[...] = dinv
    p_o[...] = dinv * xr[...]


_tc_prep = pl.pallas_call(
    _tc_prep_body,
    out_shape=[jax.ShapeDtypeStruct((ROWS_N, 128), jnp.float32)] * 2,
)


def _tc_mid_body(segp0, segp1, dinv_r, p_r, W1_r, b1_r, W2_r, q0_o, q1_o):
    dinv = dinv_r[...]
    s = dinv * (segp0[...] + segp1[...] + p_r[...])
    t0 = jnp.zeros_like(s)
    t1 = jnp.zeros_like(s)
    for j in range(16):
        h = jnp.maximum(s * W1_r[0, j] + b1_r[j], 0.0)
        t0 = t0 + h * W2_r[j, 0]
        t1 = t1 + h * W2_r[j, 1]
    q0_o[...] = dinv * t0
    q1_o[...] = dinv * t1


_tc_mid = pl.pallas_call(
    _tc_mid_body,
    in_specs=[pl.BlockSpec(memory_space=pltpu.VMEM)] * 4
    + [pl.BlockSpec(memory_space=pltpu.SMEM)] * 3,
    out_shape=[jax.ShapeDtypeStruct((ROWS_N, 128), jnp.float32)] * 2,
)


def _tc_final_body(segq0, segq1, dinv_r, q0_r, q1_r, b2_r, o0, o1):
    dinv = dinv_r[...]
    o0[...] = dinv * (segq0[...] + q0_r[...]) + b2_r[0]
    o1[...] = dinv * (segq1[...] + q1_r[...]) + b2_r[1]


_tc_final = pl.pallas_call(
    _tc_final_body,
    in_specs=[pl.BlockSpec(memory_space=pltpu.VMEM)] * 5
    + [pl.BlockSpec(memory_space=pltpu.SMEM)],
    out_shape=[jax.ShapeDtypeStruct((ROWS_N, 128), jnp.float32)] * 2,
)


def kernel(x, edge_index, W1, b1, W2, b2):
    src = edge_index[0].astype(jnp.int32)
    dst = edge_index[1].astype(jnp.int32)

    pad_n = E_PAD - E_EDGES
    padv = (N_NODES + jnp.arange(pad_n, dtype=jnp.int32) % (N_PAD - N_NODES))
    src_p = jnp.concatenate([src, padv])
    dst_p = jnp.concatenate([dst, padv])

    xpad = jnp.pad(x[:, 0], (0, N_PAD - N_NODES))
    zeros = jnp.zeros((N_PAD,), jnp.float32)
    ones = jnp.ones((WIN,), jnp.float32)

    cnt = _sc_count(dst_p, zeros, ones)
    dinv, p = _tc_prep(cnt[0].reshape(ROWS_N, 128),
                       cnt[1].reshape(ROWS_N, 128),
                       xpad.reshape(ROWS_N, 128))
    segp = _sc_seg_scalar(src_p, dst_p, p.reshape(N_PAD), zeros)
    q0, q1 = _tc_mid(segp[0].reshape(ROWS_N, 128),
                     segp[1].reshape(ROWS_N, 128),
                     dinv, p, W1, b1, W2)
    qtab = jnp.stack([q0.reshape(N_PAD), q1.reshape(N_PAD)])
    segq = _sc_seg_feat(src_p, dst_p, qtab, zeros)
    o0, o1 = _tc_final(segq[0].reshape(ROWS_N, 128),
                       segq[1].reshape(ROWS_N, 128),
                       dinv, q0, q1, b2)
    return jnp.stack([o0.reshape(N_PAD)[:N_NODES],
                      o1.reshape(N_PAD)[:N_NODES]], axis=1)


# speedup vs baseline: 1.5367x; 1.5367x over previous
import functools

import jax
import jax.numpy as jnp
from jax import lax
from jax.experimental import pallas as pl
from jax.experimental.pallas import tpu as pltpu
from jax.experimental.pallas import tpu_sc as plsc

N_NODES = 100000
N_PAD = 100352
ROWS_N = 784
E_EDGES = 6400000
E_PAD = 6553600
WIN = 8192
WING = 6400
NC, NS = 2, 16
NW = NC * NS

_mesh = plsc.VectorSubcoreMesh(
    core_axis_name="c", subcore_axis_name="s", num_cores=NC, num_subcores=NS)


def _gather_loop(tab_v, idx_v, upd_v, n):
    @plsc.parallel_loop(0, n, step=16, unroll=8)
    def step(e):
        idx16 = idx_v[pl.ds(e, 16)]
        upd_v[pl.ds(e, 16)] = plsc.load_gather(tab_v, [idx16])


@functools.partial(
    pl.kernel,
    out_type=jax.ShapeDtypeStruct((NC, N_PAD), jnp.float32),
    mesh=_mesh,
    scratch_types=[
        pltpu.VMEM((WIN,), jnp.int32),
        pltpu.VMEM((WIN,), jnp.float32),
        pltpu.VMEM_SHARED((N_PAD,), jnp.float32),
    ],
)
def _sc_count(dst_hbm, zeros_hbm, ones_hbm, out_hbm, idx_v, upd_v, acc_sh):
    cid = lax.axis_index("c")
    sid = lax.axis_index("s")
    wid = sid * NC + cid

    @pl.when(sid == 0)
    def _():
        pltpu.sync_copy(zeros_hbm, acc_sh)

    pltpu.sync_copy(ones_hbm, upd_v)
    plsc.subcore_barrier()

    def body(w, carry):
        e0 = wid * (E_PAD // NW) + w * WIN
        pltpu.sync_copy(dst_hbm.at[pl.ds(e0, WIN)], idx_v)
        pltpu.sync_copy(upd_v, acc_sh.at[idx_v], add=True)
        return carry

    lax.fori_loop(0, E_PAD // NW // WIN, body, 0)
    plsc.subcore_barrier()
    sl = N_PAD // NS
    pltpu.sync_copy(acc_sh.at[pl.ds(sid * sl, sl)],
                    out_hbm.at[cid, pl.ds(sid * sl, sl)])


@functools.partial(
    pl.kernel,
    out_type=jax.ShapeDtypeStruct((NC, N_PAD), jnp.float32),
    mesh=_mesh,
    compiler_params=pltpu.CompilerParams(needs_layout_passes=False),
    scratch_types=[
        pltpu.VMEM((N_PAD,), jnp.float32),
        pltpu.VMEM((WING,), jnp.int32),
        pltpu.VMEM((WING,), jnp.int32),
        pltpu.VMEM((WING,), jnp.float32),
        pltpu.VMEM_SHARED((N_PAD,), jnp.float32),
    ],
)
def _sc_seg_scalar(src_hbm, dst_hbm, tab_hbm, zeros_hbm, out_hbm,
                   tab_v, sidx_v, didx_v, upd_v, acc_sh):
    cid = lax.axis_index("c")
    sid = lax.axis_index("s")
    wid = sid * NC + cid

    @pl.when(sid == 0)
    def _():
        pltpu.sync_copy(zeros_hbm, acc_sh)

    pltpu.sync_copy(tab_hbm, tab_v)
    plsc.subcore_barrier()

    def body(w, carry):
        e0 = wid * (E_PAD // NW) + w * WING
        pltpu.sync_copy(src_hbm.at[pl.ds(e0, WING)], sidx_v)
        pltpu.sync_copy(dst_hbm.at[pl.ds(e0, WING)], didx_v)
        _gather_loop(tab_v, sidx_v, upd_v, WING)
        pltpu.sync_copy(upd_v, acc_sh.at[didx_v], add=True)
        return carry

    lax.fori_loop(0, E_PAD // NW // WING, body, 0)
    plsc.subcore_barrier()
    sl = N_PAD // NS
    pltpu.sync_copy(acc_sh.at[pl.ds(sid * sl, sl)],
                    out_hbm.at[cid, pl.ds(sid * sl, sl)])


@functools.partial(
    pl.kernel,
    out_type=jax.ShapeDtypeStruct((NC, N_PAD), jnp.float32),
    mesh=_mesh,
    compiler_params=pltpu.CompilerParams(needs_layout_passes=False),
    scratch_types=[
        pltpu.VMEM((N_PAD,), jnp.float32),
        pltpu.VMEM((WING,), jnp.int32),
        pltpu.VMEM((WING,), jnp.int32),
        pltpu.VMEM((WING,), jnp.float32),
        pltpu.VMEM_SHARED((N_PAD,), jnp.float32),
    ],
)
def _sc_seg_feat(src_hbm, dst_hbm, qtab_hbm, zeros_hbm, out_hbm,
                 tab_v, sidx_v, didx_v, upd_v, acc_sh):
    cid = lax.axis_index("c")
    sid = lax.axis_index("s")

    @pl.when(sid == 0)
    def _():
        pltpu.sync_copy(zeros_hbm, acc_sh)

    pltpu.sync_copy(qtab_hbm.at[cid], tab_v)
    plsc.subcore_barrier()

    def body(w, carry):
        e0 = sid * (E_PAD // NS) + w * WING
        pltpu.sync_copy(src_hbm.at[pl.ds(e0, WING)], sidx_v)
        pltpu.sync_copy(dst_hbm.at[pl.ds(e0, WING)], didx_v)
        _gather_loop(tab_v, sidx_v, upd_v, WING)
        pltpu.sync_copy(upd_v, acc_sh.at[didx_v], add=True)
        return carry

    lax.fori_loop(0, E_PAD // NS // WING, body, 0)
    plsc.subcore_barrier()
    sl = N_PAD // NS
    pltpu.sync_copy(acc_sh.at[pl.ds(sid * sl, sl)],
                    out_hbm.at[cid, pl.ds(sid * sl, sl)])


def _tc_prep_body(cnt0, cnt1, xr, dinv_o, p_o):
    deg = cnt0[...] + cnt1[...] + 1.0
    dinv = lax.rsqrt(deg)
    dinv_o[...] = dinv
    p_o[...] = dinv * xr[...]


_tc_prep = pl.pallas_call(
    _tc_prep_body,
    out_shape=[jax.ShapeDtypeStruct((ROWS_N, 128), jnp.float32)] * 2,
)


def _tc_mid_body(segp0, segp1, dinv_r, p_r, W1_r, b1_r, W2_r, q0_o, q1_o):
    dinv = dinv_r[...]
    s = dinv * (segp0[...] + segp1[...] + p_r[...])
    t0 = jnp.zeros_like(s)
    t1 = jnp.zeros_like(s)
    for j in range(16):
        h = jnp.maximum(s * W1_r[0, j] + b1_r[j], 0.0)
        t0 = t0 + h * W2_r[j, 0]
        t1 = t1 + h * W2_r[j, 1]
    q0_o[...] = dinv * t0
    q1_o[...] = dinv * t1


_tc_mid = pl.pallas_call(
    _tc_mid_body,
    in_specs=[pl.BlockSpec(memory_space=pltpu.VMEM)] * 4
    + [pl.BlockSpec(memory_space=pltpu.SMEM)] * 3,
    out_shape=[jax.ShapeDtypeStruct((ROWS_N, 128), jnp.float32)] * 2,
)


def _tc_final_body(segq0, segq1, dinv_r, q0_r, q1_r, b2_r, o0, o1):
    dinv = dinv_r[...]
    o0[...] = dinv * (segq0[...] + q0_r[...]) + b2_r[0]
    o1[...] = dinv * (segq1[...] + q1_r[...]) + b2_r[1]


_tc_final = pl.pallas_call(
    _tc_final_body,
    in_specs=[pl.BlockSpec(memory_space=pltpu.VMEM)] * 5
    + [pl.BlockSpec(memory_space=pltpu.SMEM)],
    out_shape=[jax.ShapeDtypeStruct((ROWS_N, 128), jnp.float32)] * 2,
)


def kernel(x, edge_index, W1, b1, W2, b2):
    src = edge_index[0].astype(jnp.int32)
    dst = edge_index[1].astype(jnp.int32)

    pad_n = E_PAD - E_EDGES
    padv = (N_NODES + jnp.arange(pad_n, dtype=jnp.int32) % (N_PAD - N_NODES))
    src_p = jnp.concatenate([src, padv])
    dst_p = jnp.concatenate([dst, padv])

    xpad = jnp.pad(x[:, 0], (0, N_PAD - N_NODES))
    zeros = jnp.zeros((N_PAD,), jnp.float32)
    ones = jnp.ones((WIN,), jnp.float32)

    cnt = _sc_count(dst_p, zeros, ones)
    dinv, p = _tc_prep(cnt[0].reshape(ROWS_N, 128),
                       cnt[1].reshape(ROWS_N, 128),
                       xpad.reshape(ROWS_N, 128))
    segp = _sc_seg_scalar(src_p, dst_p, p.reshape(N_PAD), zeros)
    q0, q1 = _tc_mid(segp[0].reshape(ROWS_N, 128),
                     segp[1].reshape(ROWS_N, 128),
                     dinv, p, W1, b1, W2)
    qtab = jnp.stack([q0.reshape(N_PAD), q1.reshape(N_PAD)])
    segq = _sc_seg_feat(src_p, dst_p, qtab, zeros)
    o0, o1 = _tc_final(segq[0].reshape(ROWS_N, 128),
                       segq[1].reshape(ROWS_N, 128),
                       dinv, q0, q1, b2)
    return jnp.stack([o0.reshape(N_PAD)[:N_NODES],
                      o1.reshape(N_PAD)[:N_NODES]], axis=1)


# speedup vs baseline: 1.7829x; 1.1602x over previous
import functools

import jax
import jax.numpy as jnp
from jax import lax
from jax.experimental import pallas as pl
from jax.experimental.pallas import tpu as pltpu
from jax.experimental.pallas import tpu_sc as plsc

N_NODES = 100000
N_PAD = 100352
ROWS_N = 784
E_EDGES = 6400000
E_PAD = 6553600
WIN1 = 6400
WING = 3200
NC, NS = 2, 16
NW = NC * NS

_mesh = plsc.VectorSubcoreMesh(
    core_axis_name="c", subcore_axis_name="s", num_cores=NC, num_subcores=NS)


def _gather_loop(tab_v, idx_v, upd_v, n):
    @plsc.parallel_loop(0, n, step=16, unroll=8)
    def step(e):
        idx16 = idx_v[pl.ds(e, 16)]
        upd_v[pl.ds(e, 16)] = plsc.load_gather(tab_v, [idx16])


@functools.partial(
    pl.kernel,
    out_type=jax.ShapeDtypeStruct((NC, N_PAD), jnp.float32),
    mesh=_mesh,
    scratch_types=[
        pltpu.VMEM((WIN1,), jnp.int32),
        pltpu.VMEM((WIN1,), jnp.int32),
        pltpu.VMEM((WIN1,), jnp.float32),
        pltpu.VMEM_SHARED((N_PAD,), jnp.float32),
        pltpu.SemaphoreType.DMA,
    ],
)
def _sc_count(dst_hbm, zeros_hbm, ones_hbm, out_hbm,
              idx0_v, idx1_v, ones_v, acc_sh, sem):
    cid = lax.axis_index("c")
    sid = lax.axis_index("s")
    wid = sid * NC + cid
    base = wid * (E_PAD // NW)
    nwin = E_PAD // NW // WIN1

    @pl.when(sid == 0)
    def _():
        pltpu.sync_copy(zeros_hbm, acc_sh)

    pltpu.sync_copy(ones_hbm, ones_v)
    pltpu.sync_copy(dst_hbm.at[pl.ds(base, WIN1)], idx0_v)
    plsc.subcore_barrier()

    @pl.loop(0, nwin, step=2)
    def body(wb):
        for b, cur, nxt in ((0, idx0_v, idx1_v), (1, idx1_v, idx0_v)):
            w = wb + b
            d = pltpu.async_copy(ones_v, acc_sh.at[cur], sem, add=True)

            @pl.when(w + 1 < nwin)
            def _():
                pltpu.sync_copy(
                    dst_hbm.at[pl.ds(base + (w + 1) * WIN1, WIN1)], nxt)

            d.wait()

    plsc.subcore_barrier()
    sl = N_PAD // NS
    pltpu.sync_copy(acc_sh.at[pl.ds(sid * sl, sl)],
                    out_hbm.at[cid, pl.ds(sid * sl, sl)])


def _seg_sum_sweep(src_hbm, dst_hbm, acc_sh, tab_v,
                   sidx0_v, sidx1_v, didx0_v, didx1_v, upd0_v, upd1_v, sem,
                   base, nwin):
    pltpu.sync_copy(src_hbm.at[pl.ds(base, WING)], sidx0_v)
    pltpu.sync_copy(dst_hbm.at[pl.ds(base, WING)], didx0_v)
    plsc.subcore_barrier()
    _gather_loop(tab_v, sidx0_v, upd0_v, WING)

    bufs = ((0, (sidx0_v, didx0_v, upd0_v), (sidx1_v, didx1_v, upd1_v)),
            (1, (sidx1_v, didx1_v, upd1_v), (sidx0_v, didx0_v, upd0_v)))

    @pl.loop(0, nwin, step=2)
    def body(wb):
        for b, cur, nxt in bufs:
            w = wb + b
            d = pltpu.async_copy(cur[2], acc_sh.at[cur[1]], sem, add=True)

            @pl.when(w + 1 < nwin)
            def _():
                e1 = base + (w + 1) * WING
                pltpu.sync_copy(src_hbm.at[pl.ds(e1, WING)], nxt[0])
                pltpu.sync_copy(dst_hbm.at[pl.ds(e1, WING)], nxt[1])
                _gather_loop(tab_v, nxt[0], nxt[2], WING)

            d.wait()


@functools.partial(
    pl.kernel,
    out_type=jax.ShapeDtypeStruct((NC, N_PAD), jnp.float32),
    mesh=_mesh,
    compiler_params=pltpu.CompilerParams(needs_layout_passes=False),
    scratch_types=[
        pltpu.VMEM((N_PAD,), jnp.float32),
        pltpu.VMEM((WING,), jnp.int32),
        pltpu.VMEM((WING,), jnp.int32),
        pltpu.VMEM((WING,), jnp.int32),
        pltpu.VMEM((WING,), jnp.int32),
        pltpu.VMEM((WING,), jnp.float32),
        pltpu.VMEM((WING,), jnp.float32),
        pltpu.VMEM_SHARED((N_PAD,), jnp.float32),
        pltpu.SemaphoreType.DMA,
    ],
)
def _sc_seg_scalar(src_hbm, dst_hbm, tab_hbm, zeros_hbm, out_hbm,
                   tab_v, sidx0_v, sidx1_v, didx0_v, didx1_v,
                   upd0_v, upd1_v, acc_sh, sem):
    cid = lax.axis_index("c")
    sid = lax.axis_index("s")
    wid = sid * NC + cid

    @pl.when(sid == 0)
    def _():
        pltpu.sync_copy(zeros_hbm, acc_sh)

    pltpu.sync_copy(tab_hbm, tab_v)
    _seg_sum_sweep(src_hbm, dst_hbm, acc_sh, tab_v,
                   sidx0_v, sidx1_v, didx0_v, didx1_v, upd0_v, upd1_v, sem,
                   wid * (E_PAD // NW), E_PAD // NW // WING)
    plsc.subcore_barrier()
    sl = N_PAD // NS
    pltpu.sync_copy(acc_sh.at[pl.ds(sid * sl, sl)],
                    out_hbm.at[cid, pl.ds(sid * sl, sl)])


@functools.partial(
    pl.kernel,
    out_type=jax.ShapeDtypeStruct((NC, N_PAD), jnp.float32),
    mesh=_mesh,
    compiler_params=pltpu.CompilerParams(needs_layout_passes=False),
    scratch_types=[
        pltpu.VMEM((N_PAD,), jnp.float32),
        pltpu.VMEM((WING,), jnp.int32),
        pltpu.VMEM((WING,), jnp.int32),
        pltpu.VMEM((WING,), jnp.int32),
        pltpu.VMEM((WING,), jnp.int32),
        pltpu.VMEM((WING,), jnp.float32),
        pltpu.VMEM((WING,), jnp.float32),
        pltpu.VMEM_SHARED((N_PAD,), jnp.float32),
        pltpu.SemaphoreType.DMA,
    ],
)
def _sc_seg_feat(src_hbm, dst_hbm, qtab_hbm, zeros_hbm, out_hbm,
                 tab_v, sidx0_v, sidx1_v, didx0_v, didx1_v,
                 upd0_v, upd1_v, acc_sh, sem):
    cid = lax.axis_index("c")
    sid = lax.axis_index("s")

    @pl.when(sid == 0)
    def _():
        pltpu.sync_copy(zeros_hbm, acc_sh)

    pltpu.sync_copy(qtab_hbm.at[cid], tab_v)
    _seg_sum_sweep(src_hbm, dst_hbm, acc_sh, tab_v,
                   sidx0_v, sidx1_v, didx0_v, didx1_v, upd0_v, upd1_v, sem,
                   sid * (E_PAD // NS), E_PAD // NS // WING)
    plsc.subcore_barrier()
    sl = N_PAD // NS
    pltpu.sync_copy(acc_sh.at[pl.ds(sid * sl, sl)],
                    out_hbm.at[cid, pl.ds(sid * sl, sl)])


def _tc_prep_body(cnt0, cnt1, xr, dinv_o, p_o):
    deg = cnt0[...] + cnt1[...] + 1.0
    dinv = lax.rsqrt(deg)
    dinv_o[...] = dinv
    p_o[...] = dinv * xr[...]


_tc_prep = pl.pallas_call(
    _tc_prep_body,
    out_shape=[jax.ShapeDtypeStruct((ROWS_N, 128), jnp.float32)] * 2,
)


def _tc_mid_body(segp0, segp1, dinv_r, p_r, W1_r, b1_r, W2_r, q0_o, q1_o):
    dinv = dinv_r[...]
    s = dinv * (segp0[...] + segp1[...] + p_r[...])
    t0 = jnp.zeros_like(s)
    t1 = jnp.zeros_like(s)
    for j in range(16):
        h = jnp.maximum(s * W1_r[0, j] + b1_r[j], 0.0)
        t0 = t0 + h * W2_r[j, 0]
        t1 = t1 + h * W2_r[j, 1]
    q0_o[...] = dinv * t0
    q1_o[...] = dinv * t1


_tc_mid = pl.pallas_call(
    _tc_mid_body,
    in_specs=[pl.BlockSpec(memory_space=pltpu.VMEM)] * 4
    + [pl.BlockSpec(memory_space=pltpu.SMEM)] * 3,
    out_shape=[jax.ShapeDtypeStruct((ROWS_N, 128), jnp.float32)] * 2,
)


def _tc_final_body(segq0, segq1, dinv_r, q0_r, q1_r, b2_r, o0, o1):
    dinv = dinv_r[...]
    o0[...] = dinv * (segq0[...] + q0_r[...]) + b2_r[0]
    o1[...] = dinv * (segq1[...] + q1_r[...]) + b2_r[1]


_tc_final = pl.pallas_call(
    _tc_final_body,
    in_specs=[pl.BlockSpec(memory_space=pltpu.VMEM)] * 5
    + [pl.BlockSpec(memory_space=pltpu.SMEM)],
    out_shape=[jax.ShapeDtypeStruct((ROWS_N, 128), jnp.float32)] * 2,
)


def kernel(x, edge_index, W1, b1, W2, b2):
    src = edge_index[0].astype(jnp.int32)
    dst = edge_index[1].astype(jnp.int32)

    pad_n = E_PAD - E_EDGES
    padv = (N_NODES + jnp.arange(pad_n, dtype=jnp.int32) % (N_PAD - N_NODES))
    src_p = jnp.concatenate([src, padv])
    dst_p = jnp.concatenate([dst, padv])

    xpad = jnp.pad(x[:, 0], (0, N_PAD - N_NODES))
    zeros = jnp.zeros((N_PAD,), jnp.float32)
    ones = jnp.ones((WIN1,), jnp.float32)

    cnt = _sc_count(dst_p, zeros, ones)
    dinv, p = _tc_prep(cnt[0].reshape(ROWS_N, 128),
                       cnt[1].reshape(ROWS_N, 128),
                       xpad.reshape(ROWS_N, 128))
    segp = _sc_seg_scalar(src_p, dst_p, p.reshape(N_PAD), zeros)
    q0, q1 = _tc_mid(segp[0].reshape(ROWS_N, 128),
                     segp[1].reshape(ROWS_N, 128),
                     dinv, p, W1, b1, W2)
    qtab = jnp.stack([q0.reshape(N_PAD), q1.reshape(N_PAD)])
    segq = _sc_seg_feat(src_p, dst_p, qtab, zeros)
    o0, o1 = _tc_final(segq[0].reshape(ROWS_N, 128),
                       segq[1].reshape(ROWS_N, 128),
                       dinv, q0, q1, b2)
    return jnp.stack([o0.reshape(N_PAD)[:N_NODES],
                      o1.reshape(N_PAD)[:N_NODES]], axis=1)


# speedup vs baseline: 1.9546x; 1.0963x over previous
import functools

import jax
import jax.numpy as jnp
from jax import lax
from jax.experimental import pallas as pl
from jax.experimental.pallas import tpu as pltpu
from jax.experimental.pallas import tpu_sc as plsc

N_NODES = 100000
N_PAD = 100352
ROWS_N = 784
E_EDGES = 6400000
E_PAD = 6553600
WIN1 = 6400
WING = 4096
NC, NS = 2, 16
NW = NC * NS

_mesh = plsc.VectorSubcoreMesh(
    core_axis_name="c", subcore_axis_name="s", num_cores=NC, num_subcores=NS)


def _gather_loop(tab_v, idx_v, upd_v, n):
    @plsc.parallel_loop(0, n, step=16, unroll=8)
    def step(e):
        idx16 = idx_v[pl.ds(e, 16)]
        upd_v[pl.ds(e, 16)] = plsc.load_gather(tab_v, [idx16])


@functools.partial(
    pl.kernel,
    out_type=jax.ShapeDtypeStruct((NC, N_PAD), jnp.float32),
    mesh=_mesh,
    scratch_types=[
        pltpu.VMEM((WIN1,), jnp.int32),
        pltpu.VMEM((WIN1,), jnp.int32),
        pltpu.VMEM((WIN1,), jnp.float32),
        pltpu.VMEM_SHARED((N_PAD,), jnp.float32),
        pltpu.SemaphoreType.DMA,
    ],
)
def _sc_count(dst_hbm, zeros_hbm, ones_hbm, out_hbm,
              idx0_v, idx1_v, ones_v, acc_sh, sem):
    cid = lax.axis_index("c")
    sid = lax.axis_index("s")
    wid = sid * NC + cid
    base = wid * (E_PAD // NW)
    nwin = E_PAD // NW // WIN1

    @pl.when(sid == 0)
    def _():
        pltpu.sync_copy(zeros_hbm, acc_sh)

    pltpu.sync_copy(ones_hbm, ones_v)
    pltpu.sync_copy(dst_hbm.at[pl.ds(base, WIN1)], idx0_v)
    plsc.subcore_barrier()

    @pl.loop(0, nwin, step=2)
    def body(wb):
        for b, cur, nxt in ((0, idx0_v, idx1_v), (1, idx1_v, idx0_v)):
            w = wb + b
            d = pltpu.async_copy(ones_v, acc_sh.at[cur], sem, add=True)

            @pl.when(w + 1 < nwin)
            def _():
                pltpu.sync_copy(
                    dst_hbm.at[pl.ds(base + (w + 1) * WIN1, WIN1)], nxt)

            d.wait()

    plsc.subcore_barrier()
    sl = N_PAD // NS
    pltpu.sync_copy(acc_sh.at[pl.ds(sid * sl, sl)],
                    out_hbm.at[cid, pl.ds(sid * sl, sl)])


def _seg_sum_sweep(src_hbm, dst_hbm, acc_sh, tab_v,
                   sidx0_v, sidx1_v, didx0_v, didx1_v, upd0_v, upd1_v, sem,
                   base, nwin):
    pltpu.sync_copy(src_hbm.at[pl.ds(base, WING)], sidx0_v)
    pltpu.sync_copy(dst_hbm.at[pl.ds(base, WING)], didx0_v)
    plsc.subcore_barrier()
    _gather_loop(tab_v, sidx0_v, upd0_v, WING)

    bufs = ((0, (sidx0_v, didx0_v, upd0_v), (sidx1_v, didx1_v, upd1_v)),
            (1, (sidx1_v, didx1_v, upd1_v), (sidx0_v, didx0_v, upd0_v)))

    @pl.loop(0, nwin, step=2)
    def body(wb):
        for b, cur, nxt in bufs:
            w = wb + b
            d = pltpu.async_copy(cur[2], acc_sh.at[cur[1]], sem, add=True)

            @pl.when(w + 1 < nwin)
            def _():
                e1 = base + (w + 1) * WING
                pltpu.sync_copy(src_hbm.at[pl.ds(e1, WING)], nxt[0])
                pltpu.sync_copy(dst_hbm.at[pl.ds(e1, WING)], nxt[1])
                _gather_loop(tab_v, nxt[0], nxt[2], WING)

            d.wait()


@functools.partial(
    pl.kernel,
    out_type=jax.ShapeDtypeStruct((NC, N_PAD), jnp.float32),
    mesh=_mesh,
    compiler_params=pltpu.CompilerParams(needs_layout_passes=False),
    scratch_types=[
        pltpu.VMEM((N_NODES,), jnp.float32),
        pltpu.VMEM((WING,), jnp.int32),
        pltpu.VMEM((WING,), jnp.int32),
        pltpu.VMEM((WING,), jnp.int32),
        pltpu.VMEM((WING,), jnp.int32),
        pltpu.VMEM((WING,), jnp.float32),
        pltpu.VMEM((WING,), jnp.float32),
        pltpu.VMEM_SHARED((N_PAD,), jnp.float32),
        pltpu.SemaphoreType.DMA,
    ],
)
def _sc_seg_scalar(src_hbm, dst_hbm, tab_hbm, zeros_hbm, out_hbm,
                   tab_v, sidx0_v, sidx1_v, didx0_v, didx1_v,
                   upd0_v, upd1_v, acc_sh, sem):
    cid = lax.axis_index("c")
    sid = lax.axis_index("s")
    wid = sid * NC + cid

    @pl.when(sid == 0)
    def _():
        pltpu.sync_copy(zeros_hbm, acc_sh)

    pltpu.sync_copy(tab_hbm.at[pl.ds(0, N_NODES)], tab_v)
    _seg_sum_sweep(src_hbm, dst_hbm, acc_sh, tab_v,
                   sidx0_v, sidx1_v, didx0_v, didx1_v, upd0_v, upd1_v, sem,
                   wid * (E_PAD // NW), E_PAD // NW // WING)
    plsc.subcore_barrier()
    sl = N_PAD // NS
    pltpu.sync_copy(acc_sh.at[pl.ds(sid * sl, sl)],
                    out_hbm.at[cid, pl.ds(sid * sl, sl)])


@functools.partial(
    pl.kernel,
    out_type=jax.ShapeDtypeStruct((NC, N_PAD), jnp.float32),
    mesh=_mesh,
    compiler_params=pltpu.CompilerParams(needs_layout_passes=False),
    scratch_types=[
        pltpu.VMEM((N_NODES,), jnp.float32),
        pltpu.VMEM((WING,), jnp.int32),
        pltpu.VMEM((WING,), jnp.int32),
        pltpu.VMEM((WING,), jnp.int32),
        pltpu.VMEM((WING,), jnp.int32),
        pltpu.VMEM((WING,), jnp.float32),
        pltpu.VMEM((WING,), jnp.float32),
        pltpu.VMEM_SHARED((N_PAD,), jnp.float32),
        pltpu.SemaphoreType.DMA,
    ],
)
def _sc_seg_feat(src_hbm, dst_hbm, q0tab_hbm, q1tab_hbm, zeros_hbm, out_hbm,
                 tab_v, sidx0_v, sidx1_v, didx0_v, didx1_v,
                 upd0_v, upd1_v, acc_sh, sem):
    cid = lax.axis_index("c")
    sid = lax.axis_index("s")

    @pl.when(sid == 0)
    def _():
        pltpu.sync_copy(zeros_hbm, acc_sh)

    @pl.when(cid == 0)
    def _():
        pltpu.sync_copy(q0tab_hbm.at[pl.ds(0, N_NODES)], tab_v)

    @pl.when(cid == 1)
    def _():
        pltpu.sync_copy(q1tab_hbm.at[pl.ds(0, N_NODES)], tab_v)
    _seg_sum_sweep(src_hbm, dst_hbm, acc_sh, tab_v,
                   sidx0_v, sidx1_v, didx0_v, didx1_v, upd0_v, upd1_v, sem,
                   sid * (E_PAD // NS), E_PAD // NS // WING)
    plsc.subcore_barrier()
    sl = N_PAD // NS
    pltpu.sync_copy(acc_sh.at[pl.ds(sid * sl, sl)],
                    out_hbm.at[cid, pl.ds(sid * sl, sl)])


def _tc_prep_body(cnt0, cnt1, xr, dinv_o, p_o):
    deg = cnt0[...] + cnt1[...] + 1.0
    dinv = lax.rsqrt(deg)
    dinv_o[...] = dinv
    p_o[...] = dinv * xr[...]


_tc_prep = pl.pallas_call(
    _tc_prep_body,
    out_shape=[jax.ShapeDtypeStruct((ROWS_N, 128), jnp.float32)] * 2,
)


def _tc_mid_body(segp0, segp1, dinv_r, p_r, W1_r, b1_r, W2_r, q0_o, q1_o):
    dinv = dinv_r[...]
    s = dinv * (segp0[...] + segp1[...] + p_r[...])
    t0 = jnp.zeros_like(s)
    t1 = jnp.zeros_like(s)
    for j in range(16):
        h = jnp.maximum(s * W1_r[0, j] + b1_r[j], 0.0)
        t0 = t0 + h * W2_r[j, 0]
        t1 = t1 + h * W2_r[j, 1]
    q0_o[...] = dinv * t0
    q1_o[...] = dinv * t1


_tc_mid = pl.pallas_call(
    _tc_mid_body,
    in_specs=[pl.BlockSpec(memory_space=pltpu.VMEM)] * 4
    + [pl.BlockSpec(memory_space=pltpu.SMEM)] * 3,
    out_shape=[jax.ShapeDtypeStruct((ROWS_N, 128), jnp.float32)] * 2,
)


def _tc_final_body(segq0, segq1, dinv_r, q0_r, q1_r, b2_r, o0, o1):
    dinv = dinv_r[...]
    o0[...] = dinv * (segq0[...] + q0_r[...]) + b2_r[0]
    o1[...] = dinv * (segq1[...] + q1_r[...]) + b2_r[1]


_tc_final = pl.pallas_call(
    _tc_final_body,
    in_specs=[pl.BlockSpec(memory_space=pltpu.VMEM)] * 5
    + [pl.BlockSpec(memory_space=pltpu.SMEM)],
    out_shape=[jax.ShapeDtypeStruct((ROWS_N, 128), jnp.float32)] * 2,
)


def kernel(x, edge_index, W1, b1, W2, b2):
    src = edge_index[0].astype(jnp.int32)
    dst = edge_index[1].astype(jnp.int32)

    pad_n = E_PAD - E_EDGES
    spread = jnp.arange(pad_n, dtype=jnp.int32) % (N_PAD - N_NODES)
    src_p = jnp.concatenate([src, spread])
    dst_p = jnp.concatenate([dst, N_NODES + spread])

    xpad = jnp.pad(x[:, 0], (0, N_PAD - N_NODES))
    zeros = jnp.zeros((N_PAD,), jnp.float32)
    ones = jnp.ones((WIN1,), jnp.float32)

    cnt = _sc_count(dst_p, zeros, ones)
    dinv, p = _tc_prep(cnt[0].reshape(ROWS_N, 128),
                       cnt[1].reshape(ROWS_N, 128),
                       xpad.reshape(ROWS_N, 128))
    segp = _sc_seg_scalar(src_p, dst_p, p.reshape(N_PAD), zeros)
    q0, q1 = _tc_mid(segp[0].reshape(ROWS_N, 128),
                     segp[1].reshape(ROWS_N, 128),
                     dinv, p, W1, b1, W2)
    segq = _sc_seg_feat(src_p, dst_p, q0.reshape(N_PAD), q1.reshape(N_PAD),
                        zeros)
    o0, o1 = _tc_final(segq[0].reshape(ROWS_N, 128),
                       segq[1].reshape(ROWS_N, 128),
                       dinv, q0, q1, b2)
    return jnp.stack([o0.reshape(N_PAD)[:N_NODES],
                      o1.reshape(N_PAD)[:N_NODES]], axis=1)
